# baseline (speedup 1.0000x reference)
import functools

import jax
import jax.numpy as jnp
from jax import lax
from jax.experimental import pallas as pl
from jax.experimental.pallas import tpu as pltpu
from jax.experimental.pallas import tpu_sc as plsc

_N_CODES = 8192
_DIM = 32
_ROWS = 8 * 64 * 64
_BLOCK_R = 1024
_HS = _BLOCK_R // 64
_NB = _ROWS // _BLOCK_R
_SPB = 64 // _HS
_NSPLIT = 4
_RSUB = _BLOCK_R // _NSPLIT
_BETA = 0.25


def _argmax_body(z_ref, cbt_ref, idx_ref):
    zmat = z_ref[0].reshape(_DIM, _BLOCK_R)
    z = jnp.transpose(zmat, (1, 0))
    cbt = cbt_ref[...]

    cb_norm = jnp.maximum(
        jnp.sqrt(jnp.sum(cbt * cbt, axis=0, keepdims=True)), 1e-12)
    cbn = cbt / cb_norm
    z_norm = jnp.maximum(
        jnp.sqrt(jnp.sum(z * z, axis=1, keepdims=True)), 1e-12)
    zn = z / z_norm

    parts = []
    for h in range(_NSPLIT):
        s = jax.lax.dot_general(
            zn[h * _RSUB:(h + 1) * _RSUB], cbn, (((1,), (0,)), ((), ())),
            preferred_element_type=jnp.float32)
        parts.append(jnp.argmax(s, axis=1).astype(jnp.int32))
    idx_ref[...] = jnp.concatenate(parts)


def _gather_body(table_hbm, idx_hbm, out_hbm, idx_v, rows_v, sem, *, b_per_w, nc):
    wid = lax.axis_index("s") * nc + lax.axis_index("c")
    base = wid * b_per_w
    pltpu.sync_copy(idx_hbm.at[pl.ds(base, b_per_w)], idx_v)
    pltpu.async_copy(table_hbm.at[idx_v], rows_v, sem).wait()
    pltpu.sync_copy(rows_v, out_hbm.at[pl.ds(base, b_per_w)])


def _finish_body(zq_ref, z_ref, out_ref, sse_ref):
    i = pl.program_id(0)
    zqt = jnp.transpose(zq_ref[...], (1, 0))
    zmat = z_ref[0].reshape(_DIM, _BLOCK_R)
    out_ref[...] = zqt.reshape(1, _DIM, _HS, 64)
    d = zqt - zmat

    @pl.when(i == 0)
    def _():
        sse_ref[...] = jnp.zeros_like(sse_ref)

    sse_ref[...] += jnp.sum(d * d)


def kernel(z, embed_weight):
    cbt = embed_weight.T

    encoding_indices = pl.pallas_call(
        _argmax_body,
        grid=(_NB,),
        in_specs=[
            pl.BlockSpec((1, _DIM, _HS, 64),
                         lambda i: (i // _SPB, 0, i % _SPB, 0)),
            pl.BlockSpec((_DIM, _N_CODES), lambda i: (0, 0)),
        ],
        out_specs=pl.BlockSpec((_BLOCK_R,), lambda i: (i,)),
        out_shape=jax.ShapeDtypeStruct((_ROWS,), jnp.int32),
    )(z, cbt)

    info = plsc.get_sparse_core_info()
    nc, ns = info.num_cores, info.num_subcores
    b_per_w = _ROWS // (nc * ns)

    sc_gather = pl.kernel(
        functools.partial(_gather_body, b_per_w=b_per_w, nc=nc),
        mesh=plsc.VectorSubcoreMesh(core_axis_name="c", subcore_axis_name="s"),
        compiler_params=pltpu.CompilerParams(use_tc_tiling_on_sc=False),
        out_type=jax.ShapeDtypeStruct((_ROWS, _DIM), jnp.float32),
        scratch_types=[
            pltpu.VMEM((b_per_w,), jnp.int32),
            pltpu.VMEM((b_per_w, _DIM), jnp.float32),
            pltpu.SemaphoreType.DMA,
        ],
    )
    zq_flat = sc_gather(embed_weight, encoding_indices)

    z_q_out, sse = pl.pallas_call(
        _finish_body,
        grid=(_NB,),
        in_specs=[
            pl.BlockSpec((_BLOCK_R, _DIM), lambda i: (i, 0)),
            pl.BlockSpec((1, _DIM, _HS, 64),
                         lambda i: (i // _SPB, 0, i % _SPB, 0)),
        ],
        out_specs=[
            pl.BlockSpec((1, _DIM, _HS, 64),
                         lambda i: (i // _SPB, 0, i % _SPB, 0)),
            pl.BlockSpec((1, 128), lambda i: (0, 0)),
        ],
        out_shape=[
            jax.ShapeDtypeStruct((8, _DIM, 64, 64), jnp.float32),
            jax.ShapeDtypeStruct((1, 128), jnp.float32),
        ],
    )(zq_flat, z)

    m = sse[0, 0] / jnp.float32(_ROWS * _DIM)
    loss = _BETA * m + m
    return z_q_out, loss, encoding_indices


# speedup vs baseline: 1.0850x; 1.0850x over previous
import functools

import jax
import jax.numpy as jnp
from jax import lax
from jax.experimental import pallas as pl
from jax.experimental.pallas import tpu as pltpu
from jax.experimental.pallas import tpu_sc as plsc

_N_CODES = 8192
_DIM = 32
_ROWS = 8 * 64 * 64
_BLOCK_R = 2048
_HS = _BLOCK_R // 64
_NB = _ROWS // _BLOCK_R
_SPB = 64 // _HS
_NSPLIT = 8
_RSUB = _BLOCK_R // _NSPLIT
_BETA = 0.25


def _argmax_body(z_ref, cbt_ref, idx_ref):
    zmat = z_ref[0].reshape(_DIM, _BLOCK_R)
    z = jnp.transpose(zmat, (1, 0))
    cbt = cbt_ref[...]

    cb_norm = jnp.maximum(
        jnp.sqrt(jnp.sum(cbt * cbt, axis=0, keepdims=True)), 1e-12)
    cbn = cbt / cb_norm
    z_norm = jnp.maximum(
        jnp.sqrt(jnp.sum(z * z, axis=1, keepdims=True)), 1e-12)
    zn = z / z_norm

    parts = []
    for h in range(_NSPLIT):
        s = jax.lax.dot_general(
            zn[h * _RSUB:(h + 1) * _RSUB], cbn, (((1,), (0,)), ((), ())),
            preferred_element_type=jnp.float32)
        parts.append(jnp.argmax(s, axis=1).astype(jnp.int32))
    idx_ref[...] = jnp.concatenate(parts)


def _gather_body(table_hbm, idx_hbm, out_hbm, idx_v, rows_v, sem, *, b_per_w, nc):
    wid = lax.axis_index("s") * nc + lax.axis_index("c")
    base = wid * b_per_w
    pltpu.sync_copy(idx_hbm.at[pl.ds(base, b_per_w)], idx_v)
    pltpu.async_copy(table_hbm.at[idx_v], rows_v, sem).wait()
    pltpu.sync_copy(rows_v, out_hbm.at[pl.ds(base, b_per_w)])


def _finish_body(zq_ref, z_ref, out_ref, sse_ref):
    i = pl.program_id(0)
    zqt = jnp.transpose(zq_ref[...], (1, 0))
    zmat = z_ref[0].reshape(_DIM, _BLOCK_R)
    out_ref[...] = zqt.reshape(1, _DIM, _HS, 64)
    d = zqt - zmat

    @pl.when(i == 0)
    def _():
        sse_ref[...] = jnp.zeros_like(sse_ref)

    sse_ref[...] += jnp.sum(d * d)


def kernel(z, embed_weight):
    cbt = embed_weight.T

    encoding_indices = pl.pallas_call(
        _argmax_body,
        grid=(_NB,),
        in_specs=[
            pl.BlockSpec((1, _DIM, _HS, 64),
                         lambda i: (i // _SPB, 0, i % _SPB, 0)),
            pl.BlockSpec((_DIM, _N_CODES), lambda i: (0, 0)),
        ],
        out_specs=pl.BlockSpec((_BLOCK_R,), lambda i: (i,)),
        out_shape=jax.ShapeDtypeStruct((_ROWS,), jnp.int32),
    )(z, cbt)

    info = plsc.get_sparse_core_info()
    nc, ns = info.num_cores, info.num_subcores
    b_per_w = _ROWS // (nc * ns)

    sc_gather = pl.kernel(
        functools.partial(_gather_body, b_per_w=b_per_w, nc=nc),
        mesh=plsc.VectorSubcoreMesh(core_axis_name="c", subcore_axis_name="s"),
        compiler_params=pltpu.CompilerParams(use_tc_tiling_on_sc=False),
        out_type=jax.ShapeDtypeStruct((_ROWS, _DIM), jnp.float32),
        scratch_types=[
            pltpu.VMEM((b_per_w,), jnp.int32),
            pltpu.VMEM((b_per_w, _DIM), jnp.float32),
            pltpu.SemaphoreType.DMA,
        ],
    )
    zq_flat = sc_gather(embed_weight, encoding_indices)

    z_q_out, sse = pl.pallas_call(
        _finish_body,
        grid=(_NB,),
        in_specs=[
            pl.BlockSpec((_BLOCK_R, _DIM), lambda i: (i, 0)),
            pl.BlockSpec((1, _DIM, _HS, 64),
                         lambda i: (i // _SPB, 0, i % _SPB, 0)),
        ],
        out_specs=[
            pl.BlockSpec((1, _DIM, _HS, 64),
                         lambda i: (i // _SPB, 0, i % _SPB, 0)),
            pl.BlockSpec((1, 128), lambda i: (0, 0)),
        ],
        out_shape=[
            jax.ShapeDtypeStruct((8, _DIM, 64, 64), jnp.float32),
            jax.ShapeDtypeStruct((1, 128), jnp.float32),
        ],
    )(zq_flat, z)

    m = sse[0, 0] / jnp.float32(_ROWS * _DIM)
    loss = _BETA * m + m
    return z_q_out, loss, encoding_indices


# speedup vs baseline: 1.1454x; 1.0557x over previous
import functools

import jax
import jax.numpy as jnp
from jax import lax
from jax.experimental import pallas as pl
from jax.experimental.pallas import tpu as pltpu
from jax.experimental.pallas import tpu_sc as plsc

_N_CODES = 8192
_DIM = 32
_ROWS = 8 * 64 * 64
_BLOCK_R = 4096
_HS = _BLOCK_R // 64
_NB = _ROWS // _BLOCK_R
_SPB = 64 // _HS
_NSPLIT = 16
_RSUB = _BLOCK_R // _NSPLIT
_BETA = 0.25


def _argmax_body(z_ref, cbt_ref, idx_ref):
    zmat = z_ref[0].reshape(_DIM, _BLOCK_R)
    z = jnp.transpose(zmat, (1, 0))
    cbt = cbt_ref[...]

    cb_norm = jnp.maximum(
        jnp.sqrt(jnp.sum(cbt * cbt, axis=0, keepdims=True)), 1e-12)
    cbn = cbt / cb_norm
    z_norm = jnp.maximum(
        jnp.sqrt(jnp.sum(z * z, axis=1, keepdims=True)), 1e-12)
    zn = z / z_norm

    parts = []
    for h in range(_NSPLIT):
        s = jax.lax.dot_general(
            zn[h * _RSUB:(h + 1) * _RSUB], cbn, (((1,), (0,)), ((), ())),
            preferred_element_type=jnp.float32)
        parts.append(jnp.argmax(s, axis=1).astype(jnp.int32))
    idx_ref[...] = jnp.concatenate(parts)


def _gather_body(table_hbm, idx_hbm, out_hbm, idx_v, rows_v, sem, *, b_per_w, nc):
    wid = lax.axis_index("s") * nc + lax.axis_index("c")
    base = wid * b_per_w
    pltpu.sync_copy(idx_hbm.at[pl.ds(base, b_per_w)], idx_v)
    pltpu.async_copy(table_hbm.at[idx_v], rows_v, sem).wait()
    pltpu.sync_copy(rows_v, out_hbm.at[pl.ds(base, b_per_w)])


def _finish_body(zq_ref, z_ref, out_ref, sse_ref):
    i = pl.program_id(0)
    zqt = jnp.transpose(zq_ref[...], (1, 0))
    zmat = z_ref[0].reshape(_DIM, _BLOCK_R)
    out_ref[...] = zqt.reshape(1, _DIM, _HS, 64)
    d = zqt - zmat

    @pl.when(i == 0)
    def _():
        sse_ref[...] = jnp.zeros_like(sse_ref)

    sse_ref[...] += jnp.sum(d * d)


def kernel(z, embed_weight):
    cbt = embed_weight.T

    encoding_indices = pl.pallas_call(
        _argmax_body,
        grid=(_NB,),
        in_specs=[
            pl.BlockSpec((1, _DIM, _HS, 64),
                         lambda i: (i // _SPB, 0, i % _SPB, 0)),
            pl.BlockSpec((_DIM, _N_CODES), lambda i: (0, 0)),
        ],
        out_specs=pl.BlockSpec((_BLOCK_R,), lambda i: (i,)),
        out_shape=jax.ShapeDtypeStruct((_ROWS,), jnp.int32),
    )(z, cbt)

    info = plsc.get_sparse_core_info()
    nc, ns = info.num_cores, info.num_subcores
    b_per_w = _ROWS // (nc * ns)

    sc_gather = pl.kernel(
        functools.partial(_gather_body, b_per_w=b_per_w, nc=nc),
        mesh=plsc.VectorSubcoreMesh(core_axis_name="c", subcore_axis_name="s"),
        compiler_params=pltpu.CompilerParams(use_tc_tiling_on_sc=False),
        out_type=jax.ShapeDtypeStruct((_ROWS, _DIM), jnp.float32),
        scratch_types=[
            pltpu.VMEM((b_per_w,), jnp.int32),
            pltpu.VMEM((b_per_w, _DIM), jnp.float32),
            pltpu.SemaphoreType.DMA,
        ],
    )
    zq_flat = sc_gather(embed_weight, encoding_indices)

    z_q_out, sse = pl.pallas_call(
        _finish_body,
        grid=(_NB,),
        in_specs=[
            pl.BlockSpec((_BLOCK_R, _DIM), lambda i: (i, 0)),
            pl.BlockSpec((1, _DIM, _HS, 64),
                         lambda i: (i // _SPB, 0, i % _SPB, 0)),
        ],
        out_specs=[
            pl.BlockSpec((1, _DIM, _HS, 64),
                         lambda i: (i // _SPB, 0, i % _SPB, 0)),
            pl.BlockSpec((1, 128), lambda i: (0, 0)),
        ],
        out_shape=[
            jax.ShapeDtypeStruct((8, _DIM, 64, 64), jnp.float32),
            jax.ShapeDtypeStruct((1, 128), jnp.float32),
        ],
    )(zq_flat, z)

    m = sse[0, 0] / jnp.float32(_ROWS * _DIM)
    loss = _BETA * m + m
    return z_q_out, loss, encoding_indices
